# baseline (speedup 1.0000x reference)
import jax
import jax.numpy as jnp
from jax.experimental import pallas as pl

N = 100000
D = 512
H = 256
G = 64
BLK = 10000
NBLK = N // BLK
NCHUNK = 5
CHUNK = BLK // NCHUNK


def _fused_kernel(x_ref, hub_ref, batch_ref, w1_ref, b1_ref, w2_ref, b2_ref,
                  out_ref):
    i = pl.program_id(0)
    w1_bf = w1_ref[...].astype(jnp.bfloat16)
    w2_bf = w2_ref[...].astype(jnp.bfloat16)
    b1_row = b1_ref[...][None, :].astype(jnp.bfloat16)
    partials = []
    for c in range(NCHUNK):
        x_blk = x_ref[pl.ds(c * CHUNK, CHUNK), :]
        h = jnp.dot(x_blk, w1_ref[...], preferred_element_type=jnp.float32,
                    precision=jax.lax.Precision.DEFAULT)
        h = jnp.maximum(h.astype(jnp.bfloat16) + b1_row,
                        jnp.bfloat16(0.0))
        logits = jax.lax.dot_general(
            w2_bf, h, dimension_numbers=(((0,), (1,)), ((), ())),
            preferred_element_type=jnp.float32)
        logits = logits + b2_ref[0]
        w_row = jax.nn.sigmoid(logits) * (
            1.0 + 2.0 * hub_ref[0, :, pl.ds(c * CHUNK, CHUNK)])
        gids = jax.lax.broadcasted_iota(jnp.int32, (G, CHUNK), 0)
        at = jnp.where(batch_ref[0, :, pl.ds(c * CHUNK, CHUNK)] == gids,
                       w_row, 0.0)
        partials.append(jax.lax.dot_general(
            at, x_blk,
            dimension_numbers=(((1,), (0,)), ((), ())),
            preferred_element_type=jnp.float32,
            precision=jax.lax.Precision.DEFAULT))
    partial = sum(partials[1:], start=partials[0])

    @pl.when(i == 0)
    def _init():
        out_ref[...] = partial

    @pl.when(i != 0)
    def _acc():
        out_ref[...] += partial


@jax.jit
def kernel(x, hub_scores, batch, W1, b1, W2, b2):
    hub3 = hub_scores.reshape(NBLK, 1, BLK)
    batch3 = batch.astype(jnp.int32).reshape(NBLK, 1, BLK)
    out = pl.pallas_call(
        _fused_kernel,
        grid=(NBLK,),
        in_specs=[
            pl.BlockSpec((BLK, D), lambda i: (i, 0)),
            pl.BlockSpec((1, 1, BLK), lambda i: (i, 0, 0)),
            pl.BlockSpec((1, 1, BLK), lambda i: (i, 0, 0)),
            pl.BlockSpec((D, H), lambda i: (0, 0)),
            pl.BlockSpec((H,), lambda i: (0,)),
            pl.BlockSpec((H, 1), lambda i: (0, 0)),
            pl.BlockSpec((1,), lambda i: (0,)),
        ],
        out_specs=pl.BlockSpec((G, D), lambda i: (0, 0)),
        out_shape=jax.ShapeDtypeStruct((G, D), jnp.float32),
    )(x, hub3, batch3, W1, b1, W2, b2)
    return out


# speedup vs baseline: 1.0029x; 1.0029x over previous
import jax
import jax.numpy as jnp
from jax.experimental import pallas as pl

N = 100000
D = 512
H = 256
G = 64
BLK = 10000
NBLK = N // BLK
NCHUNK = 5
CHUNK = BLK // NCHUNK


def _fused_kernel(x_ref, hub_ref, batch_ref, w1_ref, b1_ref, w2_ref, b2_ref,
                  out_ref):
    i = pl.program_id(0)
    w2_bf = w2_ref[...].astype(jnp.bfloat16)
    b1_row = b1_ref[...][None, :].astype(jnp.bfloat16)
    partials = []
    for c in range(NCHUNK):
        x_blk = x_ref[pl.ds(c * CHUNK, CHUNK), :]
        h = jnp.dot(x_blk, w1_ref[...], preferred_element_type=jnp.float32,
                    precision=jax.lax.Precision.DEFAULT)
        h = jnp.maximum(h.astype(jnp.bfloat16) + b1_row,
                        jnp.bfloat16(0.0))
        logits = jax.lax.dot_general(
            w2_bf, h, dimension_numbers=(((0,), (1,)), ((), ())),
            preferred_element_type=jnp.float32)
        logits = logits + b2_ref[0]
        w_row = jax.nn.sigmoid(logits) * (
            1.0 + 2.0 * hub_ref[0, :, pl.ds(c * CHUNK, CHUNK)])
        gids = jax.lax.broadcasted_iota(jnp.int32, (G, CHUNK), 0)
        at = jnp.where(batch_ref[0, :, pl.ds(c * CHUNK, CHUNK)] == gids,
                       w_row, 0.0)
        partials.append(jax.lax.dot_general(
            at, x_blk,
            dimension_numbers=(((1,), (0,)), ((), ())),
            preferred_element_type=jnp.float32,
            precision=jax.lax.Precision.DEFAULT))
    partial = sum(partials[1:], start=partials[0])

    @pl.when(i == 0)
    def _init():
        out_ref[...] = partial

    @pl.when(i != 0)
    def _acc():
        out_ref[...] += partial


@jax.jit
def kernel(x, hub_scores, batch, W1, b1, W2, b2):
    hub3 = hub_scores.reshape(NBLK, 1, BLK)
    batch3 = batch.astype(jnp.int32).reshape(NBLK, 1, BLK)
    out = pl.pallas_call(
        _fused_kernel,
        grid=(NBLK,),
        in_specs=[
            pl.BlockSpec((BLK, D), lambda i: (i, 0)),
            pl.BlockSpec((1, 1, BLK), lambda i: (i, 0, 0)),
            pl.BlockSpec((1, 1, BLK), lambda i: (i, 0, 0)),
            pl.BlockSpec((D, H), lambda i: (0, 0)),
            pl.BlockSpec((H,), lambda i: (0,)),
            pl.BlockSpec((H, 1), lambda i: (0, 0)),
            pl.BlockSpec((1,), lambda i: (0,)),
        ],
        out_specs=pl.BlockSpec((G, D), lambda i: (0, 0)),
        out_shape=jax.ShapeDtypeStruct((G, D), jnp.float32),
    )(x, hub3, batch3, W1, b1, W2, b2)
    return out
